# baseline (speedup 1.0000x reference)
import functools

import jax
import jax.numpy as jnp
from jax import lax
from jax.experimental import pallas as pl
from jax.experimental.pallas import tpu as pltpu
from jax.experimental.pallas import tpu_sc as plsc

BATCH = 16384
DIM = 768
GATHER_CHUNK = 64
MM_BLOCK = 2048
SC_ROWS = 10240
OH_BLOCK = 2048
NLPAD = 1024


def _sc_dims():
    try:
        info = plsc.get_sparse_core_info()
        return info.num_cores, info.num_subcores
    except Exception:
        return 2, 16


def _make_gather(num_cores, num_subcores, sc_rows, batch, dim):
    num_workers = num_cores * num_subcores
    per_worker = sc_rows // num_workers
    chunk = min(GATHER_CHUNK, per_worker)
    nchunks = per_worker // chunk
    mesh = plsc.VectorSubcoreMesh(core_axis_name="c", subcore_axis_name="s")

    @functools.partial(
        pl.kernel,
        mesh=mesh,
        out_type=jax.ShapeDtypeStruct((batch, dim), jnp.float32),
        scratch_types=[
            pltpu.VMEM((nchunks, chunk), jnp.int32),
            pltpu.VMEM((chunk, dim), jnp.float32),
            pltpu.VMEM((chunk, dim), jnp.float32),
            pltpu.SemaphoreType.DMA,
            pltpu.SemaphoreType.DMA,
            pltpu.SemaphoreType.DMA,
            pltpu.SemaphoreType.DMA,
        ],
    )
    def gather_kernel(table_hbm, idx_hbm, out_hbm, idx_v, rows0, rows1,
                      sg0, sg1, sw0, sw1):
        wid = lax.axis_index("s") * num_cores + lax.axis_index("c")
        base = wid * per_worker
        pltpu.sync_copy(idx_hbm.at[wid], idx_v)

        rows = (rows0, rows1)
        sg = (sg0, sg1)
        sw = (sw0, sw1)

        def gather(j):
            return pltpu.async_copy(
                table_hbm.at[idx_v.at[j]], rows[j % 2], sg[j % 2])

        def writeback(j):
            return pltpu.async_copy(
                rows[j % 2], out_hbm.at[pl.ds(base + j * chunk, chunk)],
                sw[j % 2])

        pending_g = {0: gather(0)}
        pending_w = {}
        for j in range(nchunks):
            if j + 1 < nchunks:
                if j - 1 >= 0:
                    pending_w.pop(j - 1).wait()
                pending_g[j + 1] = gather(j + 1)
            pending_g.pop(j).wait()
            pending_w[j] = writeback(j)
        for j in sorted(pending_w):
            pending_w.pop(j).wait()

    return gather_kernel


def _mm_body(x_ref, w_ref, b_ref, o_ref):
    o_ref[...] = (
        lax.dot_general(
            x_ref[...],
            w_ref[...],
            dimension_numbers=(((1,), (1,)), ((), ())),
            preferred_element_type=jnp.float32,
        )
        + b_ref[...]
    )


def _linear(x, w, b2d):
    batch, dim = x.shape
    grid = batch // MM_BLOCK
    return pl.pallas_call(
        _mm_body,
        grid=(grid,),
        in_specs=[
            pl.BlockSpec((MM_BLOCK, dim), lambda i: (i, 0)),
            pl.BlockSpec((dim, dim), lambda i: (0, 0)),
            pl.BlockSpec((1, dim), lambda i: (0, 0)),
        ],
        out_specs=pl.BlockSpec((MM_BLOCK, dim), lambda i: (i, 0)),
        out_shape=jax.ShapeDtypeStruct((batch, dim), jnp.float32),
    )(x, w, b2d)


def _onehot_body(txt_ref, lbl_ref, tbl_ref, o_ref):
    del txt_ref
    lbl = lbl_ref[0, 0, :]
    iota = lax.broadcasted_iota(jnp.int32, (OH_BLOCK, NLPAD), 1)
    onehot = (iota == lbl[:, None]).astype(jnp.bfloat16)
    o_ref[...] = jnp.dot(onehot, tbl_ref[...],
                         preferred_element_type=jnp.float32)


def _onehot_tail(sc_out, lbl3d, tblb, sc_rows):
    batch, dim = sc_out.shape
    tc_rows = batch - sc_rows
    return pl.pallas_call(
        _onehot_body,
        grid=(tc_rows // OH_BLOCK,),
        in_specs=[
            pl.BlockSpec(memory_space=pl.ANY),
            pl.BlockSpec((1, 1, OH_BLOCK), lambda i: (i, 0, 0)),
            pl.BlockSpec((NLPAD, dim), lambda i: (0, 0)),
        ],
        out_specs=pl.BlockSpec(
            (OH_BLOCK, dim), lambda i: (sc_rows // OH_BLOCK + i, 0)),
        out_shape=jax.ShapeDtypeStruct((batch, dim), jnp.float32),
        input_output_aliases={0: 0},
    )(sc_out, lbl3d, tblb)


def kernel(inputs, labels, W, b, text_embeds):
    num_cores, num_subcores = _sc_dims()
    gather_fn = _make_gather(num_cores, num_subcores, SC_ROWS, BATCH, DIM)
    labels32 = labels.astype(jnp.int32)
    idx3d = labels32[:SC_ROWS].reshape(
        num_cores * num_subcores, -1, GATHER_CHUNK)
    sc_out = gather_fn(text_embeds, idx3d)
    lbl3d = labels32[SC_ROWS:].reshape(
        (BATCH - SC_ROWS) // OH_BLOCK, 1, OH_BLOCK)
    tblb = jnp.pad(text_embeds.astype(jnp.bfloat16),
                   ((0, NLPAD - text_embeds.shape[0]), (0, 0)))
    text_outputs = _onehot_tail(sc_out, lbl3d, tblb, SC_ROWS)
    image_outputs = _linear(inputs, W, b.reshape(1, DIM))
    return (image_outputs, text_outputs)


# speedup vs baseline: 1.0299x; 1.0299x over previous
import functools

import jax
import jax.numpy as jnp
from jax import lax
from jax.experimental import pallas as pl
from jax.experimental.pallas import tpu as pltpu
from jax.experimental.pallas import tpu_sc as plsc

BATCH = 16384
DIM = 768
GATHER_CHUNK = 64
MM_BLOCK = 2048
SC_ROWS = 12288
OH_BLOCK = 2048
NUM_LBL = 1000


def _sc_dims():
    try:
        info = plsc.get_sparse_core_info()
        return info.num_cores, info.num_subcores
    except Exception:
        return 2, 16


def _make_gather(num_cores, num_subcores, sc_rows, batch, dim):
    num_workers = num_cores * num_subcores
    per_worker = sc_rows // num_workers
    chunk = min(GATHER_CHUNK, per_worker)
    nchunks = per_worker // chunk
    mesh = plsc.VectorSubcoreMesh(core_axis_name="c", subcore_axis_name="s")

    @functools.partial(
        pl.kernel,
        mesh=mesh,
        out_type=jax.ShapeDtypeStruct((batch, dim), jnp.float32),
        scratch_types=[
            pltpu.VMEM((nchunks, chunk), jnp.int32),
            pltpu.VMEM((chunk, dim), jnp.float32),
            pltpu.VMEM((chunk, dim), jnp.float32),
            pltpu.SemaphoreType.DMA,
            pltpu.SemaphoreType.DMA,
            pltpu.SemaphoreType.DMA,
            pltpu.SemaphoreType.DMA,
        ],
    )
    def gather_kernel(table_hbm, idx_hbm, out_hbm, idx_v, rows0, rows1,
                      sg0, sg1, sw0, sw1):
        wid = lax.axis_index("s") * num_cores + lax.axis_index("c")
        base = wid * per_worker
        pltpu.sync_copy(idx_hbm.at[wid], idx_v)

        rows = (rows0, rows1)
        sg = (sg0, sg1)
        sw = (sw0, sw1)

        def gather(j):
            return pltpu.async_copy(
                table_hbm.at[idx_v.at[j]], rows[j % 2], sg[j % 2])

        def writeback(j):
            return pltpu.async_copy(
                rows[j % 2], out_hbm.at[pl.ds(base + j * chunk, chunk)],
                sw[j % 2])

        pending_g = {0: gather(0)}
        pending_w = {}
        for j in range(nchunks):
            if j + 1 < nchunks:
                if j - 1 >= 0:
                    pending_w.pop(j - 1).wait()
                pending_g[j + 1] = gather(j + 1)
            pending_g.pop(j).wait()
            pending_w[j] = writeback(j)
        for j in sorted(pending_w):
            pending_w.pop(j).wait()

    return gather_kernel


def _mm_body(x_ref, w_ref, b_ref, o_ref):
    o_ref[...] = (
        lax.dot_general(
            x_ref[...],
            w_ref[...],
            dimension_numbers=(((1,), (1,)), ((), ())),
            preferred_element_type=jnp.float32,
        )
        + b_ref[...]
    )


def _linear(x, w, b2d):
    batch, dim = x.shape
    grid = batch // MM_BLOCK
    return pl.pallas_call(
        _mm_body,
        grid=(grid,),
        in_specs=[
            pl.BlockSpec((MM_BLOCK, dim), lambda i: (i, 0)),
            pl.BlockSpec((dim, dim), lambda i: (0, 0)),
            pl.BlockSpec((1, dim), lambda i: (0, 0)),
        ],
        out_specs=pl.BlockSpec((MM_BLOCK, dim), lambda i: (i, 0)),
        out_shape=jax.ShapeDtypeStruct((batch, dim), jnp.float32),
    )(x, w, b2d)


def _onehot_body(txt_ref, lbl_ref, tbl_ref, o_ref):
    del txt_ref
    lbl = lbl_ref[0, 0, :]
    iota = lax.broadcasted_iota(jnp.int32, (OH_BLOCK, NUM_LBL), 1)
    onehot = (iota == lbl[:, None]).astype(jnp.bfloat16)
    o_ref[...] = jnp.dot(onehot, tbl_ref[...].astype(jnp.bfloat16),
                         preferred_element_type=jnp.float32)


def _onehot_tail(sc_out, lbl3d, table, sc_rows):
    batch, dim = sc_out.shape
    tc_rows = batch - sc_rows
    return pl.pallas_call(
        _onehot_body,
        grid=(tc_rows // OH_BLOCK,),
        in_specs=[
            pl.BlockSpec(memory_space=pl.ANY),
            pl.BlockSpec((1, 1, OH_BLOCK),
                         lambda i: (sc_rows // OH_BLOCK + i, 0, 0)),
            pl.BlockSpec((NUM_LBL, dim), lambda i: (0, 0)),
        ],
        out_specs=pl.BlockSpec(
            (OH_BLOCK, dim), lambda i: (sc_rows // OH_BLOCK + i, 0)),
        out_shape=jax.ShapeDtypeStruct((batch, dim), jnp.float32),
        input_output_aliases={0: 0},
    )(sc_out, lbl3d, table)


def kernel(inputs, labels, W, b, text_embeds):
    num_cores, num_subcores = _sc_dims()
    gather_fn = _make_gather(num_cores, num_subcores, SC_ROWS, BATCH, DIM)
    labels32 = labels.astype(jnp.int32)
    idx3d = labels32[:SC_ROWS].reshape(
        num_cores * num_subcores, -1, GATHER_CHUNK)
    sc_out = gather_fn(text_embeds, idx3d)
    lbl3d = labels32.reshape(BATCH // OH_BLOCK, 1, OH_BLOCK)
    text_outputs = _onehot_tail(sc_out, lbl3d, text_embeds, SC_ROWS)
    image_outputs = _linear(inputs, W, b.reshape(1, DIM))
    return (image_outputs, text_outputs)


# speedup vs baseline: 1.0300x; 1.0001x over previous
import functools

import jax
import jax.numpy as jnp
from jax import lax
from jax.experimental import pallas as pl
from jax.experimental.pallas import tpu as pltpu
from jax.experimental.pallas import tpu_sc as plsc

BATCH = 16384
DIM = 768
GATHER_CHUNK = 64
MM_BLOCK = 2048
SC_ROWS = 14336
OH_BLOCK = 2048
NUM_LBL = 1000


def _sc_dims():
    try:
        info = plsc.get_sparse_core_info()
        return info.num_cores, info.num_subcores
    except Exception:
        return 2, 16


def _make_gather(num_cores, num_subcores, sc_rows, batch, dim):
    num_workers = num_cores * num_subcores
    per_worker = sc_rows // num_workers
    chunk = min(GATHER_CHUNK, per_worker)
    nchunks = per_worker // chunk
    mesh = plsc.VectorSubcoreMesh(core_axis_name="c", subcore_axis_name="s")

    @functools.partial(
        pl.kernel,
        mesh=mesh,
        out_type=jax.ShapeDtypeStruct((batch, dim), jnp.float32),
        scratch_types=[
            pltpu.VMEM((nchunks, chunk), jnp.int32),
            pltpu.VMEM((chunk, dim), jnp.float32),
            pltpu.VMEM((chunk, dim), jnp.float32),
            pltpu.SemaphoreType.DMA,
            pltpu.SemaphoreType.DMA,
            pltpu.SemaphoreType.DMA,
            pltpu.SemaphoreType.DMA,
        ],
    )
    def gather_kernel(table_hbm, idx_hbm, out_hbm, idx_v, rows0, rows1,
                      sg0, sg1, sw0, sw1):
        wid = lax.axis_index("s") * num_cores + lax.axis_index("c")
        base = wid * per_worker
        pltpu.sync_copy(idx_hbm.at[wid], idx_v)

        rows = (rows0, rows1)
        sg = (sg0, sg1)
        sw = (sw0, sw1)

        def gather(j):
            return pltpu.async_copy(
                table_hbm.at[idx_v.at[j]], rows[j % 2], sg[j % 2])

        def writeback(j):
            return pltpu.async_copy(
                rows[j % 2], out_hbm.at[pl.ds(base + j * chunk, chunk)],
                sw[j % 2])

        pending_g = {0: gather(0)}
        pending_w = {}
        for j in range(nchunks):
            if j + 1 < nchunks:
                if j - 1 >= 0:
                    pending_w.pop(j - 1).wait()
                pending_g[j + 1] = gather(j + 1)
            pending_g.pop(j).wait()
            pending_w[j] = writeback(j)
        for j in sorted(pending_w):
            pending_w.pop(j).wait()

    return gather_kernel


def _mm_body(x_ref, w_ref, b_ref, o_ref):
    o_ref[...] = (
        lax.dot_general(
            x_ref[...],
            w_ref[...],
            dimension_numbers=(((1,), (1,)), ((), ())),
            preferred_element_type=jnp.float32,
        )
        + b_ref[...]
    )


def _linear(x, w, b2d):
    batch, dim = x.shape
    grid = batch // MM_BLOCK
    return pl.pallas_call(
        _mm_body,
        grid=(grid,),
        in_specs=[
            pl.BlockSpec((MM_BLOCK, dim), lambda i: (i, 0)),
            pl.BlockSpec((dim, dim), lambda i: (0, 0)),
            pl.BlockSpec((1, dim), lambda i: (0, 0)),
        ],
        out_specs=pl.BlockSpec((MM_BLOCK, dim), lambda i: (i, 0)),
        out_shape=jax.ShapeDtypeStruct((batch, dim), jnp.float32),
    )(x, w, b2d)


def _onehot_body(txt_ref, lbl_ref, tbl_ref, o_ref):
    del txt_ref
    lbl = lbl_ref[0, :]
    iota = lax.broadcasted_iota(jnp.int32, (OH_BLOCK, NUM_LBL), 1)
    onehot = (iota == lbl[:, None]).astype(jnp.bfloat16)
    o_ref[...] = jnp.dot(onehot, tbl_ref[...].astype(jnp.bfloat16),
                         preferred_element_type=jnp.float32)


def _onehot_tail(sc_out, lbl3d, table, sc_rows):
    batch, dim = sc_out.shape
    tc_rows = batch - sc_rows
    return pl.pallas_call(
        _onehot_body,
        grid=(tc_rows // OH_BLOCK,),
        in_specs=[
            pl.BlockSpec(memory_space=pl.ANY),
            pl.BlockSpec((1, OH_BLOCK),
                         lambda i: (0, sc_rows // OH_BLOCK + i)),
            pl.BlockSpec((NUM_LBL, dim), lambda i: (0, 0)),
        ],
        out_specs=pl.BlockSpec(
            (OH_BLOCK, dim), lambda i: (sc_rows // OH_BLOCK + i, 0)),
        out_shape=jax.ShapeDtypeStruct((batch, dim), jnp.float32),
        input_output_aliases={0: 0},
    )(sc_out, lbl3d, table)


def kernel(inputs, labels, W, b, text_embeds):
    num_cores, num_subcores = _sc_dims()
    gather_fn = _make_gather(num_cores, num_subcores, SC_ROWS, BATCH, DIM)
    labels32 = labels.astype(jnp.int32)
    idx3d = labels32[:SC_ROWS].reshape(
        num_cores * num_subcores, -1, GATHER_CHUNK)
    sc_out = gather_fn(text_embeds, idx3d)
    lbl2d = labels32.reshape(1, BATCH)
    text_outputs = _onehot_tail(sc_out, lbl2d, text_embeds, SC_ROWS)
    image_outputs = _linear(inputs, W, b.reshape(1, DIM))
    return (image_outputs, text_outputs)


# speedup vs baseline: 1.0303x; 1.0003x over previous
import functools

import jax
import jax.numpy as jnp
from jax import lax
from jax.experimental import pallas as pl
from jax.experimental.pallas import tpu as pltpu
from jax.experimental.pallas import tpu_sc as plsc

BATCH = 16384
DIM = 768
GATHER_CHUNK = 32
MM_BLOCK = 2048
SC_ROWS = 14336
OH_BLOCK = 2048
NUM_LBL = 1000


def _sc_dims():
    try:
        info = plsc.get_sparse_core_info()
        return info.num_cores, info.num_subcores
    except Exception:
        return 2, 16


def _make_gather(num_cores, num_subcores, sc_rows, batch, dim):
    num_workers = num_cores * num_subcores
    per_worker = sc_rows // num_workers
    chunk = min(GATHER_CHUNK, per_worker)
    nchunks = per_worker // chunk
    mesh = plsc.VectorSubcoreMesh(core_axis_name="c", subcore_axis_name="s")

    @functools.partial(
        pl.kernel,
        mesh=mesh,
        out_type=jax.ShapeDtypeStruct((batch, dim), jnp.float32),
        scratch_types=[
            pltpu.VMEM((nchunks, chunk), jnp.int32),
            pltpu.VMEM((chunk, dim), jnp.float32),
            pltpu.VMEM((chunk, dim), jnp.float32),
            pltpu.VMEM_SHARED((num_subcores, 2, chunk, dim), jnp.float32),
            pltpu.SemaphoreType.DMA,
            pltpu.SemaphoreType.DMA,
            pltpu.SemaphoreType.DMA,
            pltpu.SemaphoreType.DMA,
            pltpu.SemaphoreType.DMA,
            pltpu.SemaphoreType.DMA,
        ],
    )
    def gather_kernel(table_hbm, idx_hbm, out_hbm, idx_v, rows0, rows1,
                      spmem, sg0, sg1, sh0, sh1, s20, s21):
        wid = lax.axis_index("s") * num_cores + lax.axis_index("c")
        sid = lax.axis_index("s")
        base = wid * per_worker
        pltpu.sync_copy(idx_hbm.at[wid], idx_v)

        rows = (rows0, rows1)
        sg = (sg0, sg1)
        sh = (sh0, sh1)
        s2 = (s20, s21)

        def gather(j):
            return pltpu.async_copy(
                table_hbm.at[idx_v.at[j]], rows[j % 2], sg[j % 2])

        def hop1(j):
            return pltpu.async_copy(
                rows[j % 2], spmem.at[sid, j % 2], sh[j % 2])

        def hop2(j):
            return pltpu.async_copy(
                spmem.at[sid, j % 2],
                out_hbm.at[pl.ds(base + j * chunk, chunk)],
                s2[j % 2])

        gd, h1d, h2d = {0: gather(0)}, {}, {}
        for j in range(nchunks):
            gd.pop(j).wait()
            if j - 2 >= 0:
                h2d.pop(j - 2).wait()
            h1d[j] = hop1(j)
            if j + 1 < nchunks:
                if j - 1 >= 0 and (j - 1) in h1d:
                    h1d.pop(j - 1).wait()
                gd[j + 1] = gather(j + 1)
            h1d.pop(j).wait()
            h2d[j] = hop2(j)
        for j in sorted(h2d):
            h2d.pop(j).wait()

    return gather_kernel


def _mm_body(x_ref, w_ref, b_ref, o_ref):
    o_ref[...] = (
        lax.dot_general(
            x_ref[...],
            w_ref[...],
            dimension_numbers=(((1,), (1,)), ((), ())),
            preferred_element_type=jnp.float32,
        )
        + b_ref[...]
    )


def _linear(x, w, b2d):
    batch, dim = x.shape
    grid = batch // MM_BLOCK
    return pl.pallas_call(
        _mm_body,
        grid=(grid,),
        in_specs=[
            pl.BlockSpec((MM_BLOCK, dim), lambda i: (i, 0)),
            pl.BlockSpec((dim, dim), lambda i: (0, 0)),
            pl.BlockSpec((1, dim), lambda i: (0, 0)),
        ],
        out_specs=pl.BlockSpec((MM_BLOCK, dim), lambda i: (i, 0)),
        out_shape=jax.ShapeDtypeStruct((batch, dim), jnp.float32),
    )(x, w, b2d)


def _onehot_body(txt_ref, lbl_ref, tbl_ref, o_ref):
    del txt_ref
    lbl = lbl_ref[0, :]
    iota = lax.broadcasted_iota(jnp.int32, (OH_BLOCK, NUM_LBL), 1)
    onehot = (iota == lbl[:, None]).astype(jnp.bfloat16)
    o_ref[...] = jnp.dot(onehot, tbl_ref[...].astype(jnp.bfloat16),
                         preferred_element_type=jnp.float32)


def _onehot_tail(sc_out, lbl3d, table, sc_rows):
    batch, dim = sc_out.shape
    tc_rows = batch - sc_rows
    return pl.pallas_call(
        _onehot_body,
        grid=(tc_rows // OH_BLOCK,),
        in_specs=[
            pl.BlockSpec(memory_space=pl.ANY),
            pl.BlockSpec((1, OH_BLOCK),
                         lambda i: (0, sc_rows // OH_BLOCK + i)),
            pl.BlockSpec((NUM_LBL, dim), lambda i: (0, 0)),
        ],
        out_specs=pl.BlockSpec(
            (OH_BLOCK, dim), lambda i: (sc_rows // OH_BLOCK + i, 0)),
        out_shape=jax.ShapeDtypeStruct((batch, dim), jnp.float32),
        input_output_aliases={0: 0},
    )(sc_out, lbl3d, table)


def kernel(inputs, labels, W, b, text_embeds):
    num_cores, num_subcores = _sc_dims()
    gather_fn = _make_gather(num_cores, num_subcores, SC_ROWS, BATCH, DIM)
    labels32 = labels.astype(jnp.int32)
    idx3d = labels32[:SC_ROWS].reshape(
        num_cores * num_subcores, -1, GATHER_CHUNK)
    sc_out = gather_fn(text_embeds, idx3d)
    lbl2d = labels32.reshape(1, BATCH)
    text_outputs = _onehot_tail(sc_out, lbl2d, text_embeds, SC_ROWS)
    image_outputs = _linear(inputs, W, b.reshape(1, DIM))
    return (image_outputs, text_outputs)
